# Initial kernel scaffold; baseline (speedup 1.0000x reference)
import jax
import jax.numpy as jnp
from jax.experimental import pallas as pl


def kernel(x, edge_index, batch, W1, b1, Wc1, bc1, Wc2, bc2, Wc3, bc3):
    raise NotImplementedError("write your pallas kernel here")


# speedup vs baseline: 27.4592x; 27.4592x over previous
import functools

import jax
import jax.numpy as jnp
from jax import lax
from jax.experimental import pallas as pl
from jax.experimental.pallas import tpu as pltpu
from jax.experimental.pallas import tpu_sc as plsc

N = 10000
E = 320000
D = 128
H = 32
G = 64

NC = 2
NS = 16
NW = NC * NS
EPT = E // NW
CH = 125
NCHUNK = EPT // CH
NPAD = 10240
NPT = NPAD // NS
DW = 16


def _mesh():
    return plsc.VectorSubcoreMesh(
        core_axis_name="c", subcore_axis_name="s",
        num_cores=NC, num_subcores=NS)


_SC_PARAMS = pltpu.CompilerParams(use_tc_tiling_on_sc=False)


def _deg_body(dst_hbm, ones_hbm, zeros_hbm, out_hbm, dst_v, ones_v, deg_sh):
    c = lax.axis_index("c")
    s = lax.axis_index("s")
    wid = c * NS + s
    pltpu.sync_copy(dst_hbm.at[wid], dst_v)
    pltpu.sync_copy(ones_hbm, ones_v)
    pltpu.sync_copy(zeros_hbm, deg_sh.at[pl.ds(s * NPT, NPT)])
    plsc.subcore_barrier()

    def chunk_body(j, carry):
        pltpu.sync_copy(ones_v, deg_sh.at[dst_v.at[j]], add=True)
        return carry
    lax.fori_loop(0, NCHUNK, chunk_body, 0)

    plsc.subcore_barrier()
    pltpu.sync_copy(deg_sh.at[pl.ds(s * NPT, NPT)],
                    out_hbm.at[c, pl.ds(s * NPT, NPT)])


@functools.lru_cache(maxsize=None)
def _deg_kernel():
    return pl.kernel(
        _deg_body,
        out_type=jax.ShapeDtypeStruct((NC, NPAD, DW), jnp.float32),
        mesh=_mesh(),
        compiler_params=_SC_PARAMS,
        scratch_types=[
            pltpu.VMEM((NCHUNK, CH), jnp.int32),
            pltpu.VMEM((CH, DW), jnp.float32),
            pltpu.VMEM_SHARED((NPAD, DW), jnp.float32),
        ],
    )


def _mm_body(x_ref, w_ref, degp_ref, hs_ref, dis_ref):
    deg = degp_ref[0, :, 0:1] + degp_ref[1, :, 0:1] + 1.0
    disc = lax.rsqrt(deg)
    h = jnp.dot(x_ref[...], w_ref[...], preferred_element_type=jnp.float32)
    hs_ref[...] = h * disc
    dis_ref[...] = disc


def _mm(x, w1, degp):
    return pl.pallas_call(
        _mm_body,
        out_shape=[
            jax.ShapeDtypeStruct((NPAD, H), jnp.float32),
            jax.ShapeDtypeStruct((NPAD, 1), jnp.float32),
        ],
    )(x, w1, degp)


def _agg_body(src_hbm, dst_hbm, hs_hbm, zeros_hbm, out_hbm,
              src_v, dst_v, rows_v, sem, acc_sh):
    c = lax.axis_index("c")
    s = lax.axis_index("s")
    wid = c * NS + s
    pltpu.sync_copy(src_hbm.at[wid], src_v)
    pltpu.sync_copy(dst_hbm.at[wid], dst_v)
    pltpu.sync_copy(zeros_hbm, acc_sh.at[pl.ds(s * NPT, NPT)])
    plsc.subcore_barrier()

    def chunk_body(j, carry):
        pltpu.async_copy(hs_hbm.at[src_v.at[j]], rows_v, sem).wait()
        pltpu.sync_copy(rows_v, acc_sh.at[dst_v.at[j]], add=True)
        return carry
    lax.fori_loop(0, NCHUNK, chunk_body, 0)

    plsc.subcore_barrier()
    pltpu.sync_copy(acc_sh.at[pl.ds(s * NPT, NPT)],
                    out_hbm.at[c, pl.ds(s * NPT, NPT)])


@functools.lru_cache(maxsize=None)
def _agg_kernel():
    return pl.kernel(
        _agg_body,
        out_type=jax.ShapeDtypeStruct((NC, NPAD, H), jnp.float32),
        mesh=_mesh(),
        compiler_params=_SC_PARAMS,
        scratch_types=[
            pltpu.VMEM((NCHUNK, CH), jnp.int32),
            pltpu.VMEM((NCHUNK, CH), jnp.int32),
            pltpu.VMEM((CH, H), jnp.float32),
            pltpu.SemaphoreType.DMA,
            pltpu.VMEM_SHARED((NPAD, H), jnp.float32),
        ],
    )


def _head_body(p_ref, hs_ref, dis_ref, b1_ref, batch_ref,
               wc1_ref, bc1_ref, wc2_ref, bc2_ref, wc3_ref, bc3_ref,
               out_ref, pooled_scr):
    acc = p_ref[0] + p_ref[1] + hs_ref[...]
    conv = acc * dis_ref[...] + b1_ref[...]
    batch = batch_ref[...]
    neg = jnp.full((), -jnp.inf, jnp.float32)

    def seg_body(g, carry):
        vals = jnp.where(batch == g, conv, neg)
        pooled_scr[pl.ds(g, 1), :] = jnp.max(vals, axis=0, keepdims=True)
        return carry
    lax.fori_loop(0, G, seg_body, 0)

    pooled = pooled_scr[...]
    pooled = jnp.where(jnp.isneginf(pooled), 0.0, pooled)
    z = jnp.maximum(
        jnp.dot(pooled, wc1_ref[...], preferred_element_type=jnp.float32)
        + bc1_ref[...], 0.0)
    z = jnp.maximum(
        jnp.dot(z, wc2_ref[...], preferred_element_type=jnp.float32)
        + bc2_ref[...], 0.0)
    out_ref[...] = (
        jnp.dot(z, wc3_ref[...], preferred_element_type=jnp.float32)
        + bc3_ref[...])


def _head(parts, hs, dis, b1, batch, wc1, bc1, wc2, bc2, wc3, bc3):
    return pl.pallas_call(
        _head_body,
        out_shape=jax.ShapeDtypeStruct((G, 4), jnp.float32),
        scratch_shapes=[pltpu.VMEM((G, H), jnp.float32)],
    )(parts, hs, dis, b1, batch, wc1, bc1, wc2, bc2, wc3, bc3)


def kernel(x, edge_index, batch, W1, b1, Wc1, bc1, Wc2, bc2, Wc3, bc3):
    src3 = edge_index[0].reshape(NW, NCHUNK, CH)
    dst3 = edge_index[1].reshape(NW, NCHUNK, CH)

    ones_rows = jnp.ones((CH, DW), jnp.float32)
    zeros_deg = jnp.zeros((NPT, DW), jnp.float32)
    zeros_acc = jnp.zeros((NPT, H), jnp.float32)
    x_pad = jnp.pad(x, ((0, NPAD - N), (0, 0)))
    batch_pad = jnp.pad(batch, (0, NPAD - N), constant_values=-1)

    degp = _deg_kernel()(dst3, ones_rows, zeros_deg)
    hs, dis = _mm(x_pad, W1, degp)
    parts = _agg_kernel()(src3, dst3, hs, zeros_acc)
    out = _head(parts, hs, dis,
                b1.reshape(1, H), batch_pad.reshape(NPAD, 1),
                Wc1, bc1.reshape(1, -1), Wc2, bc2.reshape(1, -1),
                Wc3, bc3.reshape(1, -1))
    return out


# speedup vs baseline: 31.7398x; 1.1559x over previous
import functools

import jax
import jax.numpy as jnp
from jax import lax
from jax.experimental import pallas as pl
from jax.experimental.pallas import tpu as pltpu
from jax.experimental.pallas import tpu_sc as plsc

N = 10000
E = 320000
D = 128
H = 32
G = 64

NC = 2
NS = 16
NW = NC * NS
EPT = E // NW
CH = 125
NCHUNK = EPT // CH
NPAD = 10240
NPT = NPAD // NS
DW = 16


def _mesh():
    return plsc.VectorSubcoreMesh(
        core_axis_name="c", subcore_axis_name="s",
        num_cores=NC, num_subcores=NS)


_SC_PARAMS = pltpu.CompilerParams(use_tc_tiling_on_sc=False)


def _deg_body(dst_hbm, ones_hbm, zeros_hbm, out_hbm, dst_v, ones_v, deg_sh):
    c = lax.axis_index("c")
    s = lax.axis_index("s")
    wid = c * NS + s
    pltpu.sync_copy(dst_hbm.at[wid], dst_v)
    pltpu.sync_copy(ones_hbm, ones_v)
    pltpu.sync_copy(zeros_hbm, deg_sh.at[pl.ds(s * NPT, NPT)])
    plsc.subcore_barrier()

    def chunk_body(j, carry):
        pltpu.sync_copy(ones_v, deg_sh.at[dst_v.at[j]], add=True)
        return carry
    lax.fori_loop(0, NCHUNK, chunk_body, 0)

    plsc.subcore_barrier()
    pltpu.sync_copy(deg_sh.at[pl.ds(s * NPT, NPT)],
                    out_hbm.at[c, pl.ds(s * NPT, NPT)])


@functools.lru_cache(maxsize=None)
def _deg_kernel():
    return pl.kernel(
        _deg_body,
        out_type=jax.ShapeDtypeStruct((NC, NPAD, DW), jnp.float32),
        mesh=_mesh(),
        compiler_params=_SC_PARAMS,
        scratch_types=[
            pltpu.VMEM((NCHUNK, CH), jnp.int32),
            pltpu.VMEM((CH, DW), jnp.float32),
            pltpu.VMEM_SHARED((NPAD, DW), jnp.float32),
        ],
    )


def _mm_body(x_ref, w_ref, degp_ref, hs_ref, dis_ref):
    deg = degp_ref[0, :, 0:1] + degp_ref[1, :, 0:1] + 1.0
    disc = lax.rsqrt(deg)
    h = jnp.dot(x_ref[...], w_ref[...], preferred_element_type=jnp.float32)
    hs_ref[...] = h * disc
    dis_ref[...] = disc


def _mm(x, w1, degp):
    return pl.pallas_call(
        _mm_body,
        out_shape=[
            jax.ShapeDtypeStruct((NPAD, H), jnp.float32),
            jax.ShapeDtypeStruct((NPAD, 1), jnp.float32),
        ],
    )(x, w1, degp)


def _agg_body(src_hbm, dst_hbm, hs_hbm, zeros_hbm, out_hbm,
              src_v, dst_v, rows0_v, rows1_v, sem0, sem1, acc_sh):
    c = lax.axis_index("c")
    s = lax.axis_index("s")
    wid = c * NS + s
    pltpu.sync_copy(src_hbm.at[wid], src_v)
    pltpu.sync_copy(dst_hbm.at[wid], dst_v)
    pltpu.sync_copy(zeros_hbm, acc_sh.at[pl.ds(s * NPT, NPT)])
    plsc.subcore_barrier()

    pltpu.async_copy(hs_hbm.at[src_v.at[0]], rows0_v, sem0)

    def pair_body(i, carry):
        j0 = 2 * i
        pltpu.async_copy(hs_hbm.at[src_v.at[j0 + 1]], rows1_v, sem1)
        pltpu.make_async_copy(hs_hbm.at[src_v.at[j0]], rows0_v, sem0).wait()
        pltpu.sync_copy(rows0_v, acc_sh.at[dst_v.at[j0]], add=True)

        @pl.when(j0 + 2 < NCHUNK)
        def _():
            pltpu.async_copy(hs_hbm.at[src_v.at[j0 + 2]], rows0_v, sem0)
        pltpu.make_async_copy(hs_hbm.at[src_v.at[j0 + 1]], rows1_v,
                              sem1).wait()
        pltpu.sync_copy(rows1_v, acc_sh.at[dst_v.at[j0 + 1]], add=True)
        return carry
    lax.fori_loop(0, NCHUNK // 2, pair_body, 0)

    plsc.subcore_barrier()
    pltpu.sync_copy(acc_sh.at[pl.ds(s * NPT, NPT)],
                    out_hbm.at[c, pl.ds(s * NPT, NPT)])


@functools.lru_cache(maxsize=None)
def _agg_kernel():
    return pl.kernel(
        _agg_body,
        out_type=jax.ShapeDtypeStruct((NC, NPAD, H), jnp.float32),
        mesh=_mesh(),
        compiler_params=_SC_PARAMS,
        scratch_types=[
            pltpu.VMEM((NCHUNK, CH), jnp.int32),
            pltpu.VMEM((NCHUNK, CH), jnp.int32),
            pltpu.VMEM((CH, H), jnp.float32),
            pltpu.VMEM((CH, H), jnp.float32),
            pltpu.SemaphoreType.DMA,
            pltpu.SemaphoreType.DMA,
            pltpu.VMEM_SHARED((NPAD, H), jnp.float32),
        ],
    )


def _head_body(p_ref, hs_ref, dis_ref, b1_ref, batch_ref,
               wc1_ref, bc1_ref, wc2_ref, bc2_ref, wc3_ref, bc3_ref,
               out_ref, pooled_scr):
    acc = p_ref[0] + p_ref[1] + hs_ref[...]
    conv = acc * dis_ref[...] + b1_ref[...]
    batch = batch_ref[...]
    neg = jnp.full((), -jnp.inf, jnp.float32)

    def seg_body(g, carry):
        vals = jnp.where(batch == g, conv, neg)
        pooled_scr[pl.ds(g, 1), :] = jnp.max(vals, axis=0, keepdims=True)
        return carry
    lax.fori_loop(0, G, seg_body, 0)

    pooled = pooled_scr[...]
    pooled = jnp.where(jnp.isneginf(pooled), 0.0, pooled)
    z = jnp.maximum(
        jnp.dot(pooled, wc1_ref[...], preferred_element_type=jnp.float32)
        + bc1_ref[...], 0.0)
    z = jnp.maximum(
        jnp.dot(z, wc2_ref[...], preferred_element_type=jnp.float32)
        + bc2_ref[...], 0.0)
    out_ref[...] = (
        jnp.dot(z, wc3_ref[...], preferred_element_type=jnp.float32)
        + bc3_ref[...])


def _head(parts, hs, dis, b1, batch, wc1, bc1, wc2, bc2, wc3, bc3):
    return pl.pallas_call(
        _head_body,
        out_shape=jax.ShapeDtypeStruct((G, 4), jnp.float32),
        scratch_shapes=[pltpu.VMEM((G, H), jnp.float32)],
    )(parts, hs, dis, b1, batch, wc1, bc1, wc2, bc2, wc3, bc3)


def kernel(x, edge_index, batch, W1, b1, Wc1, bc1, Wc2, bc2, Wc3, bc3):
    src3 = edge_index[0].reshape(NW, NCHUNK, CH)
    dst3 = edge_index[1].reshape(NW, NCHUNK, CH)

    ones_rows = jnp.ones((CH, DW), jnp.float32)
    zeros_deg = jnp.zeros((NPT, DW), jnp.float32)
    zeros_acc = jnp.zeros((NPT, H), jnp.float32)
    x_pad = jnp.pad(x, ((0, NPAD - N), (0, 0)))
    batch_pad = jnp.pad(batch, (0, NPAD - N), constant_values=-1)

    degp = _deg_kernel()(dst3, ones_rows, zeros_deg)
    hs, dis = _mm(x_pad, W1, degp)
    parts = _agg_kernel()(src3, dst3, hs, zeros_acc)
    out = _head(parts, hs, dis,
                b1.reshape(1, H), batch_pad.reshape(NPAD, 1),
                Wc1, bc1.reshape(1, -1), Wc2, bc2.reshape(1, -1),
                Wc3, bc3.reshape(1, -1))
    return out


# speedup vs baseline: 50.5991x; 1.5942x over previous
import functools

import jax
import jax.numpy as jnp
from jax import lax
from jax.experimental import pallas as pl
from jax.experimental.pallas import tpu as pltpu
from jax.experimental.pallas import tpu_sc as plsc

N = 10000
E = 320000
D = 128
H = 32
G = 64

NC = 2
NS = 16
NW = NC * NS
EPT = E // NW
CH = 125
NCHUNK = EPT // CH
NPAD = 10240
NPT = NPAD // NS
DW = 16


def _mesh():
    return plsc.VectorSubcoreMesh(
        core_axis_name="c", subcore_axis_name="s",
        num_cores=NC, num_subcores=NS)


_SC_PARAMS = pltpu.CompilerParams(use_tc_tiling_on_sc=False)


def _deg_body(dst_hbm, ones_hbm, zeros_hbm, out_hbm, dst_v, ones_v, deg_sh):
    c = lax.axis_index("c")
    s = lax.axis_index("s")
    wid = c * NS + s
    pltpu.sync_copy(dst_hbm.at[wid], dst_v)
    pltpu.sync_copy(ones_hbm, ones_v)
    pltpu.sync_copy(zeros_hbm, deg_sh.at[pl.ds(s * NPT, NPT)])
    plsc.subcore_barrier()

    def chunk_body(j, carry):
        pltpu.sync_copy(ones_v, deg_sh.at[dst_v.at[j]], add=True)
        return carry
    lax.fori_loop(0, NCHUNK, chunk_body, 0)

    plsc.subcore_barrier()
    pltpu.sync_copy(deg_sh.at[pl.ds(s * NPT, NPT)],
                    out_hbm.at[c, pl.ds(s * NPT, NPT)])


@functools.lru_cache(maxsize=None)
def _deg_kernel():
    return pl.kernel(
        _deg_body,
        out_type=jax.ShapeDtypeStruct((NC, NPAD, DW), jnp.float32),
        mesh=_mesh(),
        compiler_params=_SC_PARAMS,
        scratch_types=[
            pltpu.VMEM((NCHUNK, CH), jnp.int32),
            pltpu.VMEM((CH, DW), jnp.float32),
            pltpu.VMEM_SHARED((NPAD, DW), jnp.float32),
        ],
    )


def _mm_body(x_ref, w_ref, degp_ref, hs_ref, disb_ref):
    deg = degp_ref[0, :, 0:1] + degp_ref[1, :, 0:1] + 1.0
    disc = lax.rsqrt(deg)
    h = jnp.dot(x_ref[...], w_ref[...], preferred_element_type=jnp.float32)
    hs_ref[pl.ds(0, N)] = h * disc[:N, :]
    hs_ref[pl.ds(N, NPAD - N)] = jnp.zeros((NPAD - N, H), jnp.float32)
    disb_ref[...] = jnp.broadcast_to(disc, (NPAD, H))


def _mm(x, w1, degp):
    return pl.pallas_call(
        _mm_body,
        out_shape=[
            jax.ShapeDtypeStruct((NPAD, H), jnp.float32),
            jax.ShapeDtypeStruct((NPAD, H), jnp.float32),
        ],
    )(x, w1, degp)


def _agg_body(src_hbm, dst_hbm, hs_hbm, zeros_hbm, out_hbm,
              src_v, dst_v, rows0_v, rows1_v, sem0, sem1, acc_sh):
    c = lax.axis_index("c")
    s = lax.axis_index("s")
    wid = c * NS + s
    pltpu.sync_copy(src_hbm.at[wid], src_v)
    pltpu.sync_copy(dst_hbm.at[wid], dst_v)
    pltpu.sync_copy(zeros_hbm, acc_sh.at[pl.ds(s * NPT, NPT)])
    plsc.subcore_barrier()

    pltpu.async_copy(hs_hbm.at[src_v.at[0]], rows0_v, sem0)

    def pair_body(i, carry):
        j0 = 2 * i
        pltpu.async_copy(hs_hbm.at[src_v.at[j0 + 1]], rows1_v, sem1)
        pltpu.make_async_copy(hs_hbm.at[src_v.at[j0]], rows0_v, sem0).wait()
        pltpu.sync_copy(rows0_v, acc_sh.at[dst_v.at[j0]], add=True)

        @pl.when(j0 + 2 < NCHUNK)
        def _():
            pltpu.async_copy(hs_hbm.at[src_v.at[j0 + 2]], rows0_v, sem0)
        pltpu.make_async_copy(hs_hbm.at[src_v.at[j0 + 1]], rows1_v,
                              sem1).wait()
        pltpu.sync_copy(rows1_v, acc_sh.at[dst_v.at[j0 + 1]], add=True)
        return carry
    lax.fori_loop(0, NCHUNK // 2, pair_body, 0)

    plsc.subcore_barrier()
    pltpu.sync_copy(acc_sh.at[pl.ds(s * NPT, NPT)],
                    out_hbm.at[c, pl.ds(s * NPT, NPT)])


@functools.lru_cache(maxsize=None)
def _agg_kernel():
    return pl.kernel(
        _agg_body,
        out_type=jax.ShapeDtypeStruct((NC, NPAD, H), jnp.float32),
        mesh=_mesh(),
        compiler_params=_SC_PARAMS,
        scratch_types=[
            pltpu.VMEM((NCHUNK, CH), jnp.int32),
            pltpu.VMEM((NCHUNK, CH), jnp.int32),
            pltpu.VMEM((CH, H), jnp.float32),
            pltpu.VMEM((CH, H), jnp.float32),
            pltpu.SemaphoreType.DMA,
            pltpu.SemaphoreType.DMA,
            pltpu.VMEM_SHARED((NPAD, H), jnp.float32),
        ],
    )


SLAB = NPAD // NW


def _pool_body(p_hbm, hs_hbm, disb_hbm, batch_hbm, ninf_hbm, out_hbm,
               p0_v, p1_v, hs_v, disb_v, batch_v, pooled_v):
    c = lax.axis_index("c")
    s = lax.axis_index("s")
    wid = c * NS + s
    pltpu.sync_copy(p_hbm.at[0, wid], p0_v)
    pltpu.sync_copy(p_hbm.at[1, wid], p1_v)
    pltpu.sync_copy(hs_hbm.at[wid], hs_v)
    pltpu.sync_copy(disb_hbm.at[wid], disb_v)
    pltpu.sync_copy(batch_hbm.at[wid], batch_v)
    pltpu.sync_copy(ninf_hbm, pooled_v)

    def vec_body(i, carry):
        o = i * 16
        hs_v[pl.ds(o, 16)] = (
            p0_v[pl.ds(o, 16)] + p1_v[pl.ds(o, 16)] + hs_v[pl.ds(o, 16)]
        ) * disb_v[pl.ds(o, 16)]
        return carry
    lax.fori_loop(0, SLAB * H // 16, vec_body, 0)

    def row_body(i, carry):
        gvec = batch_v[pl.ds(i * 16, 16)]
        for k in range(16):
            g = gvec[k]

            @pl.when(g >= 0)
            def _(g=g, k=k):
                b = g * H
                o = (i * 16 + k) * H
                pooled_v[pl.ds(b, 16)] = jnp.maximum(
                    pooled_v[pl.ds(b, 16)], hs_v[pl.ds(o, 16)])
                pooled_v[pl.ds(b + 16, 16)] = jnp.maximum(
                    pooled_v[pl.ds(b + 16, 16)], hs_v[pl.ds(o + 16, 16)])
        return carry
    lax.fori_loop(0, SLAB // 16, row_body, 0)

    pltpu.sync_copy(pooled_v, out_hbm.at[wid])


@functools.lru_cache(maxsize=None)
def _pool_kernel():
    return pl.kernel(
        _pool_body,
        out_type=jax.ShapeDtypeStruct((NW, G * H), jnp.float32),
        mesh=_mesh(),
        compiler_params=_SC_PARAMS,
        scratch_types=[
            pltpu.VMEM((SLAB * H,), jnp.float32),
            pltpu.VMEM((SLAB * H,), jnp.float32),
            pltpu.VMEM((SLAB * H,), jnp.float32),
            pltpu.VMEM((SLAB * H,), jnp.float32),
            pltpu.VMEM((SLAB,), jnp.int32),
            pltpu.VMEM((G * H,), jnp.float32),
        ],
    )


def _head_body(pooledp_ref, b1_ref,
               wc1_ref, bc1_ref, wc2_ref, bc2_ref, wc3_ref, bc3_ref,
               out_ref):
    m = jnp.max(pooledp_ref[...], axis=0)
    pooled = jnp.where(jnp.isneginf(m), 0.0, m + b1_ref[...])
    z = jnp.maximum(
        jnp.dot(pooled, wc1_ref[...], preferred_element_type=jnp.float32)
        + bc1_ref[...], 0.0)
    z = jnp.maximum(
        jnp.dot(z, wc2_ref[...], preferred_element_type=jnp.float32)
        + bc2_ref[...], 0.0)
    out_ref[...] = (
        jnp.dot(z, wc3_ref[...], preferred_element_type=jnp.float32)
        + bc3_ref[...])


def _head(pooledp, b1, wc1, bc1, wc2, bc2, wc3, bc3):
    return pl.pallas_call(
        _head_body,
        out_shape=jax.ShapeDtypeStruct((G, 4), jnp.float32),
    )(pooledp, b1, wc1, bc1, wc2, bc2, wc3, bc3)


def kernel(x, edge_index, batch, W1, b1, Wc1, bc1, Wc2, bc2, Wc3, bc3):
    src3 = edge_index[0].reshape(NW, NCHUNK, CH)
    dst3 = edge_index[1].reshape(NW, NCHUNK, CH)

    ones_rows = jnp.ones((CH, DW), jnp.float32)
    zeros_deg = jnp.zeros((NPT, DW), jnp.float32)
    zeros_acc = jnp.zeros((NPT, H), jnp.float32)
    ninf = jnp.full((G * H,), -jnp.inf, jnp.float32)
    batch_pad = jnp.pad(batch, (0, NPAD - N), constant_values=-1)

    degp = _deg_kernel()(dst3, ones_rows, zeros_deg)
    hs, disb = _mm(x, W1, degp)
    parts = _agg_kernel()(src3, dst3, hs, zeros_acc)
    pooledp = _pool_kernel()(
        parts.reshape(NC, NW, SLAB * H), hs.reshape(NW, SLAB * H),
        disb.reshape(NW, SLAB * H), batch_pad.reshape(NW, SLAB), ninf)
    out = _head(pooledp.reshape(NW, G, H),
                b1.reshape(1, H),
                Wc1, bc1.reshape(1, -1), Wc2, bc2.reshape(1, -1),
                Wc3, bc3.reshape(1, -1))
    return out


# speedup vs baseline: 54.1623x; 1.0704x over previous
import functools

import jax
import jax.numpy as jnp
from jax import lax
from jax.experimental import pallas as pl
from jax.experimental.pallas import tpu as pltpu
from jax.experimental.pallas import tpu_sc as plsc

N = 10000
E = 320000
D = 128
H = 32
G = 64

NC = 2
NS = 16
NW = NC * NS
EPT = E // NW
CH = 125
NCHUNK = EPT // CH
NPAD = 10240
NPT = NPAD // NS
DW = 16


def _mesh():
    return plsc.VectorSubcoreMesh(
        core_axis_name="c", subcore_axis_name="s",
        num_cores=NC, num_subcores=NS)


_SC_PARAMS = pltpu.CompilerParams(use_tc_tiling_on_sc=False)


def _deg_body(dst_hbm, ones_hbm, zeros_hbm, out_hbm, dst_v, ones_v, dsem,
              deg_sh):
    c = lax.axis_index("c")
    s = lax.axis_index("s")
    wid = c * NS + s
    pltpu.sync_copy(dst_hbm.at[wid], dst_v)
    pltpu.sync_copy(ones_hbm, ones_v)
    pltpu.sync_copy(zeros_hbm, deg_sh.at[pl.ds(s * NPT, NPT)])
    plsc.subcore_barrier()

    def chunk_body(j, carry):
        pltpu.async_copy(ones_v, deg_sh.at[dst_v.at[j]], dsem, add=True)
        return carry
    lax.fori_loop(0, NCHUNK, chunk_body, 0)

    def drain_body(j, carry):
        pltpu.make_async_copy(ones_v, deg_sh.at[dst_v.at[j]], dsem).wait()
        return carry
    lax.fori_loop(0, NCHUNK, drain_body, 0)

    plsc.subcore_barrier()
    pltpu.sync_copy(deg_sh.at[pl.ds(s * NPT, NPT)],
                    out_hbm.at[c, pl.ds(s * NPT, NPT)])


@functools.lru_cache(maxsize=None)
def _deg_kernel():
    return pl.kernel(
        _deg_body,
        out_type=jax.ShapeDtypeStruct((NC, NPAD, DW), jnp.float32),
        mesh=_mesh(),
        compiler_params=_SC_PARAMS,
        scratch_types=[
            pltpu.VMEM((NCHUNK, CH), jnp.int32),
            pltpu.VMEM((CH, DW), jnp.float32),
            pltpu.SemaphoreType.DMA,
            pltpu.VMEM_SHARED((NPAD, DW), jnp.float32),
        ],
    )


def _mm_body(x_ref, w_ref, degp_ref, hs_ref, disb_ref):
    deg = degp_ref[0, :, 0:1] + degp_ref[1, :, 0:1] + 1.0
    disc = lax.rsqrt(deg)
    h = jnp.dot(x_ref[...], w_ref[...], preferred_element_type=jnp.float32)
    hs_ref[pl.ds(0, N)] = h * disc[:N, :]
    hs_ref[pl.ds(N, NPAD - N)] = jnp.zeros((NPAD - N, H), jnp.float32)
    disb_ref[...] = jnp.broadcast_to(disc, (NPAD, H))


def _mm(x, w1, degp):
    return pl.pallas_call(
        _mm_body,
        out_shape=[
            jax.ShapeDtypeStruct((NPAD, H), jnp.float32),
            jax.ShapeDtypeStruct((NPAD, H), jnp.float32),
        ],
    )(x, w1, degp)


NBUF = 5
PD = 2


def _agg_body(src_hbm, dst_hbm, hs_hbm, zeros_hbm, out_hbm,
              src_v, dst_v, rows_v, gsems, ssems, acc_sh):
    c = lax.axis_index("c")
    s = lax.axis_index("s")
    wid = c * NS + s
    pltpu.sync_copy(src_hbm.at[wid], src_v)
    pltpu.sync_copy(dst_hbm.at[wid], dst_v)
    pltpu.sync_copy(zeros_hbm, acc_sh.at[pl.ds(s * NPT, NPT)])
    plsc.subcore_barrier()

    for b in range(PD):
        pltpu.async_copy(hs_hbm.at[src_v.at[b]], rows_v[b], gsems[b])

    def ring_body(i, carry):
        for k in range(NBUF):
            j = NBUF * i + k
            bp = (k + PD) % NBUF
            pltpu.make_async_copy(hs_hbm.at[src_v.at[j]], rows_v[k],
                                  gsems[k]).wait()
            pltpu.async_copy(rows_v[k], acc_sh.at[dst_v.at[j]], ssems[k],
                             add=True)

            @pl.when((j + PD < NCHUNK) & (j >= NBUF - PD))
            def _(j=j, bp=bp):
                pltpu.make_async_copy(rows_v[bp],
                                      acc_sh.at[dst_v.at[j - (NBUF - PD)]],
                                      ssems[bp]).wait()

            @pl.when(j + PD < NCHUNK)
            def _(j=j, bp=bp):
                pltpu.async_copy(hs_hbm.at[src_v.at[j + PD]], rows_v[bp],
                                 gsems[bp])
        return carry
    lax.fori_loop(0, NCHUNK // NBUF, ring_body, 0)

    for k in range(NBUF):
        pltpu.make_async_copy(rows_v[k],
                              acc_sh.at[dst_v.at[NCHUNK - NBUF + k]],
                              ssems[k]).wait()

    plsc.subcore_barrier()
    pltpu.sync_copy(acc_sh.at[pl.ds(s * NPT, NPT)],
                    out_hbm.at[c, pl.ds(s * NPT, NPT)])


@functools.lru_cache(maxsize=None)
def _agg_kernel():
    return pl.kernel(
        _agg_body,
        out_type=jax.ShapeDtypeStruct((NC, NPAD, H), jnp.float32),
        mesh=_mesh(),
        compiler_params=_SC_PARAMS,
        scratch_types=[
            pltpu.VMEM((NCHUNK, CH), jnp.int32),
            pltpu.VMEM((NCHUNK, CH), jnp.int32),
            [pltpu.VMEM((CH, H), jnp.float32) for _ in range(NBUF)],
            [pltpu.SemaphoreType.DMA for _ in range(NBUF)],
            [pltpu.SemaphoreType.DMA for _ in range(NBUF)],
            pltpu.VMEM_SHARED((NPAD, H), jnp.float32),
        ],
    )


SLAB = NPAD // NW


def _pool_body(p_hbm, hs_hbm, disb_hbm, batch_hbm, ninf_hbm, out_hbm,
               p0_v, p1_v, hs_v, disb_v, batch_v, pooled_v):
    c = lax.axis_index("c")
    s = lax.axis_index("s")
    wid = c * NS + s
    pltpu.sync_copy(p_hbm.at[0, wid], p0_v)
    pltpu.sync_copy(p_hbm.at[1, wid], p1_v)
    pltpu.sync_copy(hs_hbm.at[wid], hs_v)
    pltpu.sync_copy(disb_hbm.at[wid], disb_v)
    pltpu.sync_copy(batch_hbm.at[wid], batch_v)
    pltpu.sync_copy(ninf_hbm, pooled_v)

    def vec_body(i, carry):
        o = i * 16
        hs_v[pl.ds(o, 16)] = (
            p0_v[pl.ds(o, 16)] + p1_v[pl.ds(o, 16)] + hs_v[pl.ds(o, 16)]
        ) * disb_v[pl.ds(o, 16)]
        return carry
    lax.fori_loop(0, SLAB * H // 16, vec_body, 0)

    def row_body(i, carry):
        gvec = batch_v[pl.ds(i * 16, 16)]
        for k in range(16):
            g = gvec[k]

            @pl.when(g >= 0)
            def _(g=g, k=k):
                b = g * H
                o = (i * 16 + k) * H
                pooled_v[pl.ds(b, 16)] = jnp.maximum(
                    pooled_v[pl.ds(b, 16)], hs_v[pl.ds(o, 16)])
                pooled_v[pl.ds(b + 16, 16)] = jnp.maximum(
                    pooled_v[pl.ds(b + 16, 16)], hs_v[pl.ds(o + 16, 16)])
        return carry
    lax.fori_loop(0, SLAB // 16, row_body, 0)

    pltpu.sync_copy(pooled_v, out_hbm.at[wid])


@functools.lru_cache(maxsize=None)
def _pool_kernel():
    return pl.kernel(
        _pool_body,
        out_type=jax.ShapeDtypeStruct((NW, G * H), jnp.float32),
        mesh=_mesh(),
        compiler_params=_SC_PARAMS,
        scratch_types=[
            pltpu.VMEM((SLAB * H,), jnp.float32),
            pltpu.VMEM((SLAB * H,), jnp.float32),
            pltpu.VMEM((SLAB * H,), jnp.float32),
            pltpu.VMEM((SLAB * H,), jnp.float32),
            pltpu.VMEM((SLAB,), jnp.int32),
            pltpu.VMEM((G * H,), jnp.float32),
        ],
    )


def _head_body(pooledp_ref, b1_ref,
               wc1_ref, bc1_ref, wc2_ref, bc2_ref, wc3_ref, bc3_ref,
               out_ref):
    m = jnp.max(pooledp_ref[...], axis=0)
    pooled = jnp.where(jnp.isneginf(m), 0.0, m + b1_ref[...])
    z = jnp.maximum(
        jnp.dot(pooled, wc1_ref[...], preferred_element_type=jnp.float32)
        + bc1_ref[...], 0.0)
    z = jnp.maximum(
        jnp.dot(z, wc2_ref[...], preferred_element_type=jnp.float32)
        + bc2_ref[...], 0.0)
    out_ref[...] = (
        jnp.dot(z, wc3_ref[...], preferred_element_type=jnp.float32)
        + bc3_ref[...])


def _head(pooledp, b1, wc1, bc1, wc2, bc2, wc3, bc3):
    return pl.pallas_call(
        _head_body,
        out_shape=jax.ShapeDtypeStruct((G, 4), jnp.float32),
    )(pooledp, b1, wc1, bc1, wc2, bc2, wc3, bc3)


def kernel(x, edge_index, batch, W1, b1, Wc1, bc1, Wc2, bc2, Wc3, bc3):
    src3 = edge_index[0].reshape(NW, NCHUNK, CH)
    dst3 = edge_index[1].reshape(NW, NCHUNK, CH)

    ones_rows = jnp.ones((CH, DW), jnp.float32)
    zeros_deg = jnp.zeros((NPT, DW), jnp.float32)
    zeros_acc = jnp.zeros((NPT, H), jnp.float32)
    ninf = jnp.full((G * H,), -jnp.inf, jnp.float32)
    batch_pad = jnp.pad(batch, (0, NPAD - N), constant_values=-1)

    degp = _deg_kernel()(dst3, ones_rows, zeros_deg)
    hs, disb = _mm(x, W1, degp)
    parts = _agg_kernel()(src3, dst3, hs, zeros_acc)
    pooledp = _pool_kernel()(
        parts.reshape(NC, NW, SLAB * H), hs.reshape(NW, SLAB * H),
        disb.reshape(NW, SLAB * H), batch_pad.reshape(NW, SLAB), ninf)
    out = _head(pooledp.reshape(NW, G, H),
                b1.reshape(1, H),
                Wc1, bc1.reshape(1, -1), Wc2, bc2.reshape(1, -1),
                Wc3, bc3.reshape(1, -1))
    return out


# speedup vs baseline: 58.3289x; 1.0769x over previous
import functools

import jax
import jax.numpy as jnp
from jax import lax
from jax.experimental import pallas as pl
from jax.experimental.pallas import tpu as pltpu
from jax.experimental.pallas import tpu_sc as plsc

N = 10000
E = 320000
D = 128
H = 32
G = 64

NC = 2
NS = 16
NW = NC * NS
EPT = E // NW
CH = 125
NCHUNK = EPT // CH
NPAD = 10240
NPT = NPAD // NS
DW = 16


def _mesh():
    return plsc.VectorSubcoreMesh(
        core_axis_name="c", subcore_axis_name="s",
        num_cores=NC, num_subcores=NS)


_SC_PARAMS = pltpu.CompilerParams(use_tc_tiling_on_sc=False)


def _deg_body(e_hbm, ones_hbm, zeros_hbm, out_hbm, dst_v, ones_v, dsem,
              deg_sh):
    c = lax.axis_index("c")
    s = lax.axis_index("s")
    wid = c * NS + s
    pltpu.sync_copy(e_hbm.at[1, wid], dst_v)
    pltpu.sync_copy(ones_hbm, ones_v)
    pltpu.sync_copy(zeros_hbm, deg_sh.at[pl.ds(s * NPT, NPT)])
    plsc.subcore_barrier()

    def chunk_body(j, carry):
        pltpu.async_copy(ones_v, deg_sh.at[dst_v.at[j]], dsem, add=True)
        return carry
    lax.fori_loop(0, NCHUNK, chunk_body, 0)

    def drain_body(j, carry):
        pltpu.make_async_copy(ones_v, deg_sh.at[dst_v.at[j]], dsem).wait()
        return carry
    lax.fori_loop(0, NCHUNK, drain_body, 0)

    plsc.subcore_barrier()
    pltpu.sync_copy(deg_sh.at[pl.ds(s * NPT, NPT)],
                    out_hbm.at[c, pl.ds(s * NPT, NPT)])


@functools.lru_cache(maxsize=None)
def _deg_kernel():
    return pl.kernel(
        _deg_body,
        out_type=jax.ShapeDtypeStruct((NC, NPAD, DW), jnp.float32),
        mesh=_mesh(),
        compiler_params=_SC_PARAMS,
        scratch_types=[
            pltpu.VMEM((NCHUNK, CH), jnp.int32),
            pltpu.VMEM((CH, DW), jnp.float32),
            pltpu.SemaphoreType.DMA,
            pltpu.VMEM_SHARED((NPAD, DW), jnp.float32),
        ],
    )


def _mm_body(x_ref, w_ref, degp_ref, hs_ref, disb_ref):
    deg = degp_ref[0, :, 0:1] + degp_ref[1, :, 0:1] + 1.0
    disc = lax.rsqrt(deg)
    h = jnp.dot(x_ref[...], w_ref[...], preferred_element_type=jnp.float32)
    hs_ref[pl.ds(0, N)] = h * disc[:N, :]
    hs_ref[pl.ds(N, NPAD - N)] = jnp.zeros((NPAD - N, H), jnp.float32)
    disb_ref[...] = jnp.broadcast_to(disc, (NPAD, H))


def _mm(x, w1, degp):
    return pl.pallas_call(
        _mm_body,
        out_shape=[
            jax.ShapeDtypeStruct((NPAD, H), jnp.float32),
            jax.ShapeDtypeStruct((NPAD, H), jnp.float32),
        ],
    )(x, w1, degp)


NBUF = 5
PD = 2


def _agg_body(e_hbm, hs_hbm, zeros_hbm, out_hbm,
              src_v, dst_v, rows_v, gsems, ssems, acc_sh):
    c = lax.axis_index("c")
    s = lax.axis_index("s")
    wid = c * NS + s
    pltpu.sync_copy(e_hbm.at[0, wid], src_v)
    pltpu.sync_copy(e_hbm.at[1, wid], dst_v)
    pltpu.sync_copy(zeros_hbm, acc_sh.at[pl.ds(s * NPT, NPT)])
    plsc.subcore_barrier()

    for b in range(PD):
        pltpu.async_copy(hs_hbm.at[src_v.at[b]], rows_v[b], gsems[b])

    def ring_body(i, carry):
        for k in range(NBUF):
            j = NBUF * i + k
            bp = (k + PD) % NBUF
            pltpu.make_async_copy(hs_hbm.at[src_v.at[j]], rows_v[k],
                                  gsems[k]).wait()
            pltpu.async_copy(rows_v[k], acc_sh.at[dst_v.at[j]], ssems[k],
                             add=True)

            @pl.when((j + PD < NCHUNK) & (j >= NBUF - PD))
            def _(j=j, bp=bp):
                pltpu.make_async_copy(rows_v[bp],
                                      acc_sh.at[dst_v.at[j - (NBUF - PD)]],
                                      ssems[bp]).wait()

            @pl.when(j + PD < NCHUNK)
            def _(j=j, bp=bp):
                pltpu.async_copy(hs_hbm.at[src_v.at[j + PD]], rows_v[bp],
                                 gsems[bp])
        return carry
    lax.fori_loop(0, NCHUNK // NBUF, ring_body, 0)

    for k in range(NBUF):
        pltpu.make_async_copy(rows_v[k],
                              acc_sh.at[dst_v.at[NCHUNK - NBUF + k]],
                              ssems[k]).wait()

    plsc.subcore_barrier()
    pltpu.sync_copy(acc_sh.at[pl.ds(s * NPT, NPT)],
                    out_hbm.at[c, pl.ds(s * NPT, NPT)])


@functools.lru_cache(maxsize=None)
def _agg_kernel():
    return pl.kernel(
        _agg_body,
        out_type=jax.ShapeDtypeStruct((NC, NPAD, H), jnp.float32),
        mesh=_mesh(),
        compiler_params=_SC_PARAMS,
        scratch_types=[
            pltpu.VMEM((NCHUNK, CH), jnp.int32),
            pltpu.VMEM((NCHUNK, CH), jnp.int32),
            [pltpu.VMEM((CH, H), jnp.float32) for _ in range(NBUF)],
            [pltpu.SemaphoreType.DMA for _ in range(NBUF)],
            [pltpu.SemaphoreType.DMA for _ in range(NBUF)],
            pltpu.VMEM_SHARED((NPAD, H), jnp.float32),
        ],
    )


SLAB = NPAD // NW


def _pool_body(p_hbm, hs_hbm, disb_hbm, batch_hbm, ninf_hbm, out_hbm,
               p0_v, p1_v, hs_v, disb_v, batch_v, pooled_v):
    c = lax.axis_index("c")
    s = lax.axis_index("s")
    wid = c * NS + s
    base = wid * SLAB
    pltpu.sync_copy(p_hbm.at[0, pl.ds(base, SLAB)], p0_v)
    pltpu.sync_copy(p_hbm.at[1, pl.ds(base, SLAB)], p1_v)
    pltpu.sync_copy(hs_hbm.at[pl.ds(base, SLAB)], hs_v)
    pltpu.sync_copy(disb_hbm.at[pl.ds(base, SLAB)], disb_v)
    pltpu.sync_copy(batch_hbm.at[pl.ds(base, SLAB)], batch_v)
    pltpu.sync_copy(ninf_hbm, pooled_v)

    def vec_body(d, carry):
        for k in range(2):
            f = pl.ds(16 * k, 16)
            hs_v[d, f] = (
                p0_v[d, f] + p1_v[d, f] + hs_v[d, f]) * disb_v[d, f]
        return carry
    lax.fori_loop(0, SLAB, vec_body, 0)

    def row_body(i, carry):
        gvec = batch_v[pl.ds(i * 16, 16)]
        for k in range(16):
            g = gvec[k]

            @pl.when(g >= 0)
            def _(g=g, k=k):
                d = i * 16 + k
                for f in range(2):
                    sl = pl.ds(16 * f, 16)
                    pooled_v[g, sl] = jnp.maximum(pooled_v[g, sl],
                                                  hs_v[d, sl])
        return carry
    lax.fori_loop(0, SLAB // 16, row_body, 0)

    pltpu.sync_copy(pooled_v, out_hbm.at[wid])


@functools.lru_cache(maxsize=None)
def _pool_kernel():
    return pl.kernel(
        _pool_body,
        out_type=jax.ShapeDtypeStruct((NW, G, H), jnp.float32),
        mesh=_mesh(),
        compiler_params=_SC_PARAMS,
        scratch_types=[
            pltpu.VMEM((SLAB, H), jnp.float32),
            pltpu.VMEM((SLAB, H), jnp.float32),
            pltpu.VMEM((SLAB, H), jnp.float32),
            pltpu.VMEM((SLAB, H), jnp.float32),
            pltpu.VMEM((SLAB,), jnp.int32),
            pltpu.VMEM((G, H), jnp.float32),
        ],
    )


def _head_body(pooledp_ref, b1_ref,
               wc1_ref, bc1_ref, wc2_ref, bc2_ref, wc3_ref, bc3_ref,
               out_ref):
    m = jnp.max(pooledp_ref[...], axis=0)
    pooled = jnp.where(jnp.isneginf(m), 0.0, m + b1_ref[...])
    z = jnp.maximum(
        jnp.dot(pooled, wc1_ref[...], preferred_element_type=jnp.float32)
        + bc1_ref[...], 0.0)
    z = jnp.maximum(
        jnp.dot(z, wc2_ref[...], preferred_element_type=jnp.float32)
        + bc2_ref[...], 0.0)
    out_ref[...] = (
        jnp.dot(z, wc3_ref[...], preferred_element_type=jnp.float32)
        + bc3_ref[...])


def _head(pooledp, b1, wc1, bc1, wc2, bc2, wc3, bc3):
    return pl.pallas_call(
        _head_body,
        out_shape=jax.ShapeDtypeStruct((G, 4), jnp.float32),
    )(pooledp, b1, wc1, bc1, wc2, bc2, wc3, bc3)


def kernel(x, edge_index, batch, W1, b1, Wc1, bc1, Wc2, bc2, Wc3, bc3):
    e4 = edge_index.reshape(2, NW, NCHUNK, CH)

    ones_rows = jnp.ones((CH, DW), jnp.float32)
    zeros_deg = jnp.zeros((NPT, DW), jnp.float32)
    zeros_acc = jnp.zeros((NPT, H), jnp.float32)
    ninf = jnp.full((G, H), -jnp.inf, jnp.float32)
    batch_pad = jnp.pad(batch, (0, NPAD - N), constant_values=-1)

    degp = _deg_kernel()(e4, ones_rows, zeros_deg)
    hs, disb = _mm(x, W1, degp)
    parts = _agg_kernel()(e4, hs, zeros_acc)
    pooledp = _pool_kernel()(parts, hs, disb, batch_pad, ninf)
    out = _head(pooledp, b1.reshape(1, H),
                Wc1, bc1.reshape(1, -1), Wc2, bc2.reshape(1, -1),
                Wc3, bc3.reshape(1, -1))
    return out


# speedup vs baseline: 64.8649x; 1.1121x over previous
import functools

import jax
import jax.numpy as jnp
from jax import lax
from jax.experimental import pallas as pl
from jax.experimental.pallas import tpu as pltpu
from jax.experimental.pallas import tpu_sc as plsc

N = 10000
E = 320000
D = 128
H = 32
G = 64

NC = 2
NS = 16
NW = NC * NS
EPT = E // NW
CH = 125
NCHUNK = EPT // CH
NPAD = 10240
NPT = NPAD // NS
DW = 8


def _mesh():
    return plsc.VectorSubcoreMesh(
        core_axis_name="c", subcore_axis_name="s",
        num_cores=NC, num_subcores=NS)


_SC_PARAMS = pltpu.CompilerParams(use_tc_tiling_on_sc=False)


def _deg_body(e_hbm, ones_hbm, zeros_hbm, out_hbm, dst_v, ones_v, dsem,
              deg_sh):
    c = lax.axis_index("c")
    s = lax.axis_index("s")
    wid = c * NS + s
    pltpu.sync_copy(e_hbm.at[1, wid], dst_v)
    pltpu.sync_copy(ones_hbm, ones_v)
    pltpu.sync_copy(zeros_hbm, deg_sh.at[pl.ds(s * NPT, NPT)])
    plsc.subcore_barrier()

    def chunk_body(j, carry):
        pltpu.async_copy(ones_v, deg_sh.at[dst_v.at[j]], dsem, add=True)
        return carry
    lax.fori_loop(0, NCHUNK, chunk_body, 0)

    def drain_body(j, carry):
        pltpu.make_async_copy(ones_v, deg_sh.at[dst_v.at[j]], dsem).wait()
        return carry
    lax.fori_loop(0, NCHUNK, drain_body, 0)

    plsc.subcore_barrier()
    pltpu.sync_copy(deg_sh.at[pl.ds(s * NPT, NPT)],
                    out_hbm.at[c, pl.ds(s * NPT, NPT)])


@functools.lru_cache(maxsize=None)
def _deg_kernel():
    return pl.kernel(
        _deg_body,
        out_type=jax.ShapeDtypeStruct((NC, NPAD, DW), jnp.float32),
        mesh=_mesh(),
        compiler_params=_SC_PARAMS,
        scratch_types=[
            pltpu.VMEM((NCHUNK, CH), jnp.int32),
            pltpu.VMEM((CH, DW), jnp.float32),
            pltpu.SemaphoreType.DMA,
            pltpu.VMEM_SHARED((NPAD, DW), jnp.float32),
        ],
    )


def _mm_body(x_ref, w_ref, degp_ref, hs_ref, disb_ref):
    deg = degp_ref[0, :, 0:1] + degp_ref[1, :, 0:1] + 1.0
    disc = lax.rsqrt(deg)
    h = jnp.dot(x_ref[...], w_ref[...], preferred_element_type=jnp.float32)
    hs_ref[pl.ds(0, N)] = h * disc[:N, :]
    hs_ref[pl.ds(N, NPAD - N)] = jnp.zeros((NPAD - N, H), jnp.float32)
    disb_ref[...] = jnp.broadcast_to(disc, (NPAD, H))


def _mm(x, w1, degp):
    return pl.pallas_call(
        _mm_body,
        out_shape=[
            jax.ShapeDtypeStruct((NPAD, H), jnp.float32),
            jax.ShapeDtypeStruct((NPAD, H), jnp.float32),
        ],
    )(x, w1, degp)


NBUF = 8
PD = 4


def _agg_body(e_hbm, hs_hbm, zeros_hbm, out_hbm,
              src_v, dst_v, rows_v, gsems, ssems, acc_sh):
    c = lax.axis_index("c")
    s = lax.axis_index("s")
    wid = c * NS + s
    pltpu.sync_copy(e_hbm.at[0, wid], src_v)
    pltpu.sync_copy(e_hbm.at[1, wid], dst_v)
    pltpu.sync_copy(zeros_hbm, acc_sh.at[pl.ds(s * NPT, NPT)])
    plsc.subcore_barrier()

    for b in range(PD):
        pltpu.async_copy(hs_hbm.at[src_v.at[b]], rows_v[b], gsems[b])

    def ring_body(i, carry):
        for k in range(NBUF):
            j = NBUF * i + k
            bp = (k + PD) % NBUF
            pltpu.make_async_copy(hs_hbm.at[src_v.at[j]], rows_v[k],
                                  gsems[k]).wait()
            pltpu.async_copy(rows_v[k], acc_sh.at[dst_v.at[j]], ssems[k],
                             add=True)

            @pl.when((j + PD < NCHUNK) & (j >= NBUF - PD))
            def _(j=j, bp=bp):
                pltpu.make_async_copy(rows_v[bp],
                                      acc_sh.at[dst_v.at[j - (NBUF - PD)]],
                                      ssems[bp]).wait()

            @pl.when(j + PD < NCHUNK)
            def _(j=j, bp=bp):
                pltpu.async_copy(hs_hbm.at[src_v.at[j + PD]], rows_v[bp],
                                 gsems[bp])
        return carry
    lax.fori_loop(0, NCHUNK // NBUF, ring_body, 0)

    for k in range(NBUF):
        pltpu.make_async_copy(rows_v[k],
                              acc_sh.at[dst_v.at[NCHUNK - NBUF + k]],
                              ssems[k]).wait()

    plsc.subcore_barrier()
    pltpu.sync_copy(acc_sh.at[pl.ds(s * NPT, NPT)],
                    out_hbm.at[c, pl.ds(s * NPT, NPT)])


@functools.lru_cache(maxsize=None)
def _agg_kernel():
    return pl.kernel(
        _agg_body,
        out_type=jax.ShapeDtypeStruct((NC, NPAD, H), jnp.float32),
        mesh=_mesh(),
        compiler_params=_SC_PARAMS,
        scratch_types=[
            pltpu.VMEM((NCHUNK, CH), jnp.int32),
            pltpu.VMEM((NCHUNK, CH), jnp.int32),
            [pltpu.VMEM((CH, H), jnp.float32) for _ in range(NBUF)],
            [pltpu.SemaphoreType.DMA for _ in range(NBUF)],
            [pltpu.SemaphoreType.DMA for _ in range(NBUF)],
            pltpu.VMEM_SHARED((NPAD, H), jnp.float32),
        ],
    )


SLAB = NPAD // NW


def _pool_body(p_hbm, hs_hbm, disb_hbm, batch_hbm, ninf_hbm, out_hbm,
               p0_v, p1_v, hs_v, disb_v, batch_v, pooled_v):
    c = lax.axis_index("c")
    s = lax.axis_index("s")
    wid = c * NS + s
    base = wid * SLAB
    pltpu.sync_copy(p_hbm.at[0, pl.ds(base, SLAB)], p0_v)
    pltpu.sync_copy(p_hbm.at[1, pl.ds(base, SLAB)], p1_v)
    pltpu.sync_copy(hs_hbm.at[pl.ds(base, SLAB)], hs_v)
    pltpu.sync_copy(disb_hbm.at[pl.ds(base, SLAB)], disb_v)
    pltpu.sync_copy(batch_hbm.at[pl.ds(base, SLAB)], batch_v)
    pltpu.sync_copy(ninf_hbm, pooled_v)

    def vec_body(d, carry):
        for k in range(2):
            f = pl.ds(16 * k, 16)
            hs_v[d, f] = (
                p0_v[d, f] + p1_v[d, f] + hs_v[d, f]) * disb_v[d, f]
        return carry
    lax.fori_loop(0, SLAB, vec_body, 0)

    def row_body(i, carry):
        gvec = batch_v[pl.ds(i * 16, 16)]
        for k in range(16):
            g = gvec[k]

            @pl.when(g >= 0)
            def _(g=g, k=k):
                d = i * 16 + k
                for f in range(2):
                    sl = pl.ds(16 * f, 16)
                    pooled_v[g, sl] = jnp.maximum(pooled_v[g, sl],
                                                  hs_v[d, sl])
        return carry
    lax.fori_loop(0, SLAB // 16, row_body, 0)

    pltpu.sync_copy(pooled_v, out_hbm.at[wid])


@functools.lru_cache(maxsize=None)
def _pool_kernel():
    return pl.kernel(
        _pool_body,
        out_type=jax.ShapeDtypeStruct((NW, G, H), jnp.float32),
        mesh=_mesh(),
        compiler_params=_SC_PARAMS,
        scratch_types=[
            pltpu.VMEM((SLAB, H), jnp.float32),
            pltpu.VMEM((SLAB, H), jnp.float32),
            pltpu.VMEM((SLAB, H), jnp.float32),
            pltpu.VMEM((SLAB, H), jnp.float32),
            pltpu.VMEM((SLAB,), jnp.int32),
            pltpu.VMEM((G, H), jnp.float32),
        ],
    )


def _head_body(pooledp_ref, b1_ref,
               wc1_ref, bc1_ref, wc2_ref, bc2_ref, wc3_ref, bc3_ref,
               out_ref):
    m = jnp.max(pooledp_ref[...], axis=0)
    pooled = jnp.where(jnp.isneginf(m), 0.0, m + b1_ref[...])
    z = jnp.maximum(
        jnp.dot(pooled, wc1_ref[...], preferred_element_type=jnp.float32)
        + bc1_ref[...], 0.0)
    z = jnp.maximum(
        jnp.dot(z, wc2_ref[...], preferred_element_type=jnp.float32)
        + bc2_ref[...], 0.0)
    out_ref[...] = (
        jnp.dot(z, wc3_ref[...], preferred_element_type=jnp.float32)
        + bc3_ref[...])


def _head(pooledp, b1, wc1, bc1, wc2, bc2, wc3, bc3):
    return pl.pallas_call(
        _head_body,
        out_shape=jax.ShapeDtypeStruct((G, 4), jnp.float32),
    )(pooledp, b1, wc1, bc1, wc2, bc2, wc3, bc3)


def kernel(x, edge_index, batch, W1, b1, Wc1, bc1, Wc2, bc2, Wc3, bc3):
    e4 = edge_index.reshape(2, NW, NCHUNK, CH)

    ones_rows = jnp.ones((CH, DW), jnp.float32)
    zeros_deg = jnp.zeros((NPT, DW), jnp.float32)
    zeros_acc = jnp.zeros((NPT, H), jnp.float32)
    ninf = jnp.full((G, H), -jnp.inf, jnp.float32)
    batch_pad = jnp.pad(batch, (0, NPAD - N), constant_values=-1)

    degp = _deg_kernel()(e4, ones_rows, zeros_deg)
    hs, disb = _mm(x, W1, degp)
    parts = _agg_kernel()(e4, hs, zeros_acc)
    pooledp = _pool_kernel()(parts, hs, disb, batch_pad, ninf)
    out = _head(pooledp, b1.reshape(1, H),
                Wc1, bc1.reshape(1, -1), Wc2, bc2.reshape(1, -1),
                Wc3, bc3.reshape(1, -1))
    return out


# speedup vs baseline: 65.9059x; 1.0160x over previous
import functools

import jax
import jax.numpy as jnp
from jax import lax
from jax.experimental import pallas as pl
from jax.experimental.pallas import tpu as pltpu
from jax.experimental.pallas import tpu_sc as plsc

N = 10000
E = 320000
D = 128
H = 32
G = 64

NC = 2
NS = 16
NW = NC * NS
EPT = E // NW
CH = 125
NCHUNK = EPT // CH
NPAD = 10240
NPT = NPAD // NS
DW = 8


def _mesh():
    return plsc.VectorSubcoreMesh(
        core_axis_name="c", subcore_axis_name="s",
        num_cores=NC, num_subcores=NS)


_SC_PARAMS = pltpu.CompilerParams(use_tc_tiling_on_sc=False)


def _deg_body(e_hbm, ones_hbm, zeros_hbm, out_hbm, dst_v, ones_v, dsem,
              deg_sh):
    c = lax.axis_index("c")
    s = lax.axis_index("s")
    wid = c * NS + s
    pltpu.sync_copy(e_hbm.at[1, wid], dst_v)
    pltpu.sync_copy(ones_hbm, ones_v)
    pltpu.sync_copy(zeros_hbm, deg_sh.at[pl.ds(s * NPT, NPT)])
    plsc.subcore_barrier()

    def chunk_body(j, carry):
        pltpu.async_copy(ones_v, deg_sh.at[dst_v.at[j]], dsem, add=True)
        return carry
    lax.fori_loop(0, NCHUNK, chunk_body, 0)

    def drain_body(j, carry):
        pltpu.make_async_copy(ones_v, deg_sh.at[dst_v.at[j]], dsem).wait()
        return carry
    lax.fori_loop(0, NCHUNK, drain_body, 0)

    plsc.subcore_barrier()
    pltpu.sync_copy(deg_sh.at[pl.ds(s * NPT, NPT)],
                    out_hbm.at[c, pl.ds(s * NPT, NPT)])


@functools.lru_cache(maxsize=None)
def _deg_kernel():
    return pl.kernel(
        _deg_body,
        out_type=jax.ShapeDtypeStruct((NC, NPAD, DW), jnp.float32),
        mesh=_mesh(),
        compiler_params=_SC_PARAMS,
        scratch_types=[
            pltpu.VMEM((NCHUNK, CH), jnp.int32),
            pltpu.VMEM((CH, DW), jnp.float32),
            pltpu.SemaphoreType.DMA,
            pltpu.VMEM_SHARED((NPAD, DW), jnp.float32),
        ],
    )


def _mma_body(x_ref, w_ref, h_ref):
    h_ref[pl.ds(0, N)] = jnp.dot(x_ref[...], w_ref[...],
                                 preferred_element_type=jnp.float32)
    h_ref[pl.ds(N, NPAD - N)] = jnp.zeros((NPAD - N, H), jnp.float32)


def _mma(x, w1):
    return pl.pallas_call(
        _mma_body,
        out_shape=jax.ShapeDtypeStruct((NPAD, H), jnp.float32),
    )(x, w1)


def _mmb_body(h_ref, degp_ref, hs_ref, disb_ref):
    deg = degp_ref[0, :, 0:1] + degp_ref[1, :, 0:1] + 1.0
    disc = lax.rsqrt(deg)
    hs_ref[...] = h_ref[...] * disc
    disb_ref[...] = jnp.broadcast_to(disc, (NPAD, H))


def _mmb(h, degp):
    return pl.pallas_call(
        _mmb_body,
        out_shape=[
            jax.ShapeDtypeStruct((NPAD, H), jnp.float32),
            jax.ShapeDtypeStruct((NPAD, H), jnp.float32),
        ],
    )(h, degp)


NBUF = 8
PD = 4


def _agg_body(e_hbm, hs_hbm, zeros_hbm, out_hbm,
              src_v, dst_v, rows_v, gsems, ssems, acc_sh):
    c = lax.axis_index("c")
    s = lax.axis_index("s")
    wid = c * NS + s
    pltpu.sync_copy(e_hbm.at[0, wid], src_v)
    pltpu.sync_copy(e_hbm.at[1, wid], dst_v)
    pltpu.sync_copy(zeros_hbm, acc_sh.at[pl.ds(s * NPT, NPT)])
    plsc.subcore_barrier()

    for b in range(PD):
        pltpu.async_copy(hs_hbm.at[src_v.at[b]], rows_v[b], gsems[b])

    def ring_body(i, carry):
        for k in range(NBUF):
            j = NBUF * i + k
            bp = (k + PD) % NBUF
            pltpu.make_async_copy(hs_hbm.at[src_v.at[j]], rows_v[k],
                                  gsems[k]).wait()
            pltpu.async_copy(rows_v[k], acc_sh.at[dst_v.at[j]], ssems[k],
                             add=True)

            @pl.when((j + PD < NCHUNK) & (j >= NBUF - PD))
            def _(j=j, bp=bp):
                pltpu.make_async_copy(rows_v[bp],
                                      acc_sh.at[dst_v.at[j - (NBUF - PD)]],
                                      ssems[bp]).wait()

            @pl.when(j + PD < NCHUNK)
            def _(j=j, bp=bp):
                pltpu.async_copy(hs_hbm.at[src_v.at[j + PD]], rows_v[bp],
                                 gsems[bp])
        return carry
    lax.fori_loop(0, NCHUNK // NBUF, ring_body, 0)

    for k in range(NBUF):
        pltpu.make_async_copy(rows_v[k],
                              acc_sh.at[dst_v.at[NCHUNK - NBUF + k]],
                              ssems[k]).wait()

    plsc.subcore_barrier()
    pltpu.sync_copy(acc_sh.at[pl.ds(s * NPT, NPT)],
                    out_hbm.at[c, pl.ds(s * NPT, NPT)])


@functools.lru_cache(maxsize=None)
def _agg_kernel():
    return pl.kernel(
        _agg_body,
        out_type=jax.ShapeDtypeStruct((NC, NPAD, H), jnp.float32),
        mesh=_mesh(),
        compiler_params=_SC_PARAMS,
        scratch_types=[
            pltpu.VMEM((NCHUNK, CH), jnp.int32),
            pltpu.VMEM((NCHUNK, CH), jnp.int32),
            [pltpu.VMEM((CH, H), jnp.float32) for _ in range(NBUF)],
            [pltpu.SemaphoreType.DMA for _ in range(NBUF)],
            [pltpu.SemaphoreType.DMA for _ in range(NBUF)],
            pltpu.VMEM_SHARED((NPAD, H), jnp.float32),
        ],
    )


SLAB = NPAD // NW


def _pool_body(p_hbm, hs_hbm, disb_hbm, batch_hbm, ninf_hbm, out_hbm,
               p0_v, p1_v, hs_v, disb_v, batch_v, pooled_v, lsem):
    c = lax.axis_index("c")
    s = lax.axis_index("s")
    wid = c * NS + s
    base = wid * SLAB
    cps = [
        pltpu.make_async_copy(p_hbm.at[0, pl.ds(base, SLAB)], p0_v, lsem),
        pltpu.make_async_copy(p_hbm.at[1, pl.ds(base, SLAB)], p1_v, lsem),
        pltpu.make_async_copy(hs_hbm.at[pl.ds(base, SLAB)], hs_v, lsem),
        pltpu.make_async_copy(disb_hbm.at[pl.ds(base, SLAB)], disb_v, lsem),
        pltpu.make_async_copy(batch_hbm.at[pl.ds(base, SLAB)], batch_v, lsem),
        pltpu.make_async_copy(ninf_hbm, pooled_v, lsem),
    ]
    for cp in cps:
        cp.start()
    for cp in cps:
        cp.wait()

    def vec_body(d, carry):
        for k in range(2):
            f = pl.ds(16 * k, 16)
            hs_v[d, f] = (
                p0_v[d, f] + p1_v[d, f] + hs_v[d, f]) * disb_v[d, f]
        return carry
    lax.fori_loop(0, SLAB, vec_body, 0)

    def row_body(i, carry):
        gvec = batch_v[pl.ds(i * 16, 16)]
        for k in range(16):
            g = gvec[k]

            @pl.when(g >= 0)
            def _(g=g, k=k):
                d = i * 16 + k
                for f in range(2):
                    sl = pl.ds(16 * f, 16)
                    pooled_v[g, sl] = jnp.maximum(pooled_v[g, sl],
                                                  hs_v[d, sl])
        return carry
    lax.fori_loop(0, SLAB // 16, row_body, 0)

    pltpu.sync_copy(pooled_v, out_hbm.at[wid])


@functools.lru_cache(maxsize=None)
def _pool_kernel():
    return pl.kernel(
        _pool_body,
        out_type=jax.ShapeDtypeStruct((NW, G, H), jnp.float32),
        mesh=_mesh(),
        compiler_params=_SC_PARAMS,
        scratch_types=[
            pltpu.VMEM((SLAB, H), jnp.float32),
            pltpu.VMEM((SLAB, H), jnp.float32),
            pltpu.VMEM((SLAB, H), jnp.float32),
            pltpu.VMEM((SLAB, H), jnp.float32),
            pltpu.VMEM((SLAB,), jnp.int32),
            pltpu.VMEM((G, H), jnp.float32),
            pltpu.SemaphoreType.DMA,
        ],
    )


def _head_body(pooledp_ref, b1_ref,
               wc1_ref, bc1_ref, wc2_ref, bc2_ref, wc3_ref, bc3_ref,
               out_ref):
    m = jnp.max(pooledp_ref[...], axis=0)
    pooled = jnp.where(jnp.isneginf(m), 0.0, m + b1_ref[...])
    z = jnp.maximum(
        jnp.dot(pooled, wc1_ref[...], preferred_element_type=jnp.float32)
        + bc1_ref[...], 0.0)
    z = jnp.maximum(
        jnp.dot(z, wc2_ref[...], preferred_element_type=jnp.float32)
        + bc2_ref[...], 0.0)
    out_ref[...] = (
        jnp.dot(z, wc3_ref[...], preferred_element_type=jnp.float32)
        + bc3_ref[...])


def _head(pooledp, b1, wc1, bc1, wc2, bc2, wc3, bc3):
    return pl.pallas_call(
        _head_body,
        out_shape=jax.ShapeDtypeStruct((G, 4), jnp.float32),
    )(pooledp, b1, wc1, bc1, wc2, bc2, wc3, bc3)


def kernel(x, edge_index, batch, W1, b1, Wc1, bc1, Wc2, bc2, Wc3, bc3):
    e4 = edge_index.reshape(2, NW, NCHUNK, CH)

    ones_rows = jnp.ones((CH, DW), jnp.float32)
    zeros_deg = jnp.zeros((NPT, DW), jnp.float32)
    zeros_acc = jnp.zeros((NPT, H), jnp.float32)
    ninf = jnp.full((G, H), -jnp.inf, jnp.float32)
    batch_pad = jnp.pad(batch, (0, NPAD - N), constant_values=-1)

    h = _mma(x, W1)
    degp = _deg_kernel()(e4, ones_rows, zeros_deg)
    hs, disb = _mmb(h, degp)
    parts = _agg_kernel()(e4, hs, zeros_acc)
    pooledp = _pool_kernel()(parts, hs, disb, batch_pad, ninf)
    out = _head(pooledp, b1.reshape(1, H),
                Wc1, bc1.reshape(1, -1), Wc2, bc2.reshape(1, -1),
                Wc3, bc3.reshape(1, -1))
    return out


# speedup vs baseline: 74.3385x; 1.1280x over previous
import functools

import jax
import jax.numpy as jnp
from jax import lax
from jax.experimental import pallas as pl
from jax.experimental.pallas import tpu as pltpu
from jax.experimental.pallas import tpu_sc as plsc

N = 10000
E = 320000
D = 128
H = 32
G = 64

NC = 2
NS = 16
NW = NC * NS
EPT = E // NW
CH = 125
NCHUNK = EPT // CH
NPAD = 10240
NPT = NPAD // NS
DW = 16


def _mesh():
    return plsc.VectorSubcoreMesh(
        core_axis_name="c", subcore_axis_name="s",
        num_cores=NC, num_subcores=NS)


_SC_PARAMS = pltpu.CompilerParams(use_tc_tiling_on_sc=False)


def _deg_body(e_hbm, ones_hbm, zeros_hbm, out_hbm, dst_v, ones_v, dsem,
              deg_v, degb_v, deg_sh):
    c = lax.axis_index("c")
    s = lax.axis_index("s")
    wid = c * NS + s
    pltpu.sync_copy(e_hbm.at[1, wid], dst_v)
    pltpu.sync_copy(ones_hbm, ones_v)
    pltpu.sync_copy(zeros_hbm, deg_sh.at[pl.ds(s * NPT, NPT)])
    plsc.subcore_barrier()

    def chunk_body(j, carry):
        pltpu.async_copy(ones_v, deg_sh.at[dst_v.at[j]], dsem, add=True)
        return carry
    lax.fori_loop(0, NCHUNK, chunk_body, 0)

    def drain_body(j, carry):
        pltpu.make_async_copy(ones_v, deg_sh.at[dst_v.at[j]], dsem).wait()
        return carry
    lax.fori_loop(0, NCHUNK, drain_body, 0)

    plsc.subcore_barrier()
    pltpu.sync_copy(deg_sh.at[pl.ds(s * NPT, NPT)], deg_v)

    def splat_body(n, carry):
        val = deg_v[n]
        flat = n * H
        row = flat // 128
        col = flat % 128
        degb_v[row, pl.ds(col, 16)] = val
        degb_v[row, pl.ds(col + 16, 16)] = val
        return carry
    lax.fori_loop(0, NPT, splat_body, 0)
    pltpu.sync_copy(degb_v, out_hbm.at[c, pl.ds(s * (NPT * H // 128),
                                                NPT * H // 128)])


@functools.lru_cache(maxsize=None)
def _deg_kernel():
    return pl.kernel(
        _deg_body,
        out_type=jax.ShapeDtypeStruct((NC, NPAD * H // 128, 128),
                                      jnp.float32),
        mesh=_mesh(),
        compiler_params=_SC_PARAMS,
        scratch_types=[
            pltpu.VMEM((NCHUNK, CH), jnp.int32),
            pltpu.VMEM((CH, DW), jnp.float32),
            pltpu.SemaphoreType.DMA,
            pltpu.VMEM((NPT, DW), jnp.float32),
            pltpu.VMEM((NPT * H // 128, 128), jnp.float32),
            pltpu.VMEM_SHARED((NPAD, DW), jnp.float32),
        ],
    )


def _mma_body(x_ref, w_ref, h_ref):
    h_ref[pl.ds(0, N)] = jnp.dot(x_ref[...], w_ref[...],
                                 preferred_element_type=jnp.float32)
    h_ref[pl.ds(N, NPAD - N)] = jnp.zeros((NPAD - N, H), jnp.float32)


def _mma(x, w1):
    return pl.pallas_call(
        _mma_body,
        out_shape=jax.ShapeDtypeStruct((NPAD, H), jnp.float32),
    )(x, w1)


NF = NPAD * H // 128


def _mmb_body(h_ref, degb_ref, hs_ref, disb_ref):
    deg = degb_ref[0] + degb_ref[1] + 1.0
    disc = lax.rsqrt(deg)
    hs_ref[...] = h_ref[...] * disc
    disb_ref[...] = disc


def _mmb(h4, degb):
    return pl.pallas_call(
        _mmb_body,
        out_shape=[
            jax.ShapeDtypeStruct((NF, 128), jnp.float32),
            jax.ShapeDtypeStruct((NF, 128), jnp.float32),
        ],
    )(h4, degb)


NBUF = 8
PD = 4


def _agg_body(e_hbm, hs_hbm, zeros_hbm, out_hbm,
              src_v, dst_v, rows_v, gsems, ssems, acc_sh):
    c = lax.axis_index("c")
    s = lax.axis_index("s")
    wid = c * NS + s
    pltpu.sync_copy(e_hbm.at[0, wid], src_v)
    pltpu.sync_copy(e_hbm.at[1, wid], dst_v)
    pltpu.sync_copy(zeros_hbm, acc_sh.at[pl.ds(s * NPT, NPT)])
    plsc.subcore_barrier()

    for b in range(PD):
        pltpu.async_copy(hs_hbm.at[src_v.at[b]], rows_v[b], gsems[b])

    def ring_body(i, carry):
        for k in range(NBUF):
            j = NBUF * i + k
            bp = (k + PD) % NBUF
            pltpu.make_async_copy(hs_hbm.at[src_v.at[j]], rows_v[k],
                                  gsems[k]).wait()
            pltpu.async_copy(rows_v[k], acc_sh.at[dst_v.at[j]], ssems[k],
                             add=True)

            @pl.when((j + PD < NCHUNK) & (j >= NBUF - PD))
            def _(j=j, bp=bp):
                pltpu.make_async_copy(rows_v[bp],
                                      acc_sh.at[dst_v.at[j - (NBUF - PD)]],
                                      ssems[bp]).wait()

            @pl.when(j + PD < NCHUNK)
            def _(j=j, bp=bp):
                pltpu.async_copy(hs_hbm.at[src_v.at[j + PD]], rows_v[bp],
                                 gsems[bp])
        return carry
    lax.fori_loop(0, NCHUNK // NBUF, ring_body, 0)

    for k in range(NBUF):
        pltpu.make_async_copy(rows_v[k],
                              acc_sh.at[dst_v.at[NCHUNK - NBUF + k]],
                              ssems[k]).wait()

    plsc.subcore_barrier()
    pltpu.sync_copy(acc_sh.at[pl.ds(s * NPT, NPT)],
                    out_hbm.at[c, pl.ds(s * NPT, NPT)])


@functools.lru_cache(maxsize=None)
def _agg_kernel():
    return pl.kernel(
        _agg_body,
        out_type=jax.ShapeDtypeStruct((NC, NPAD, H), jnp.float32),
        mesh=_mesh(),
        compiler_params=_SC_PARAMS,
        scratch_types=[
            pltpu.VMEM((NCHUNK, CH), jnp.int32),
            pltpu.VMEM((NCHUNK, CH), jnp.int32),
            [pltpu.VMEM((CH, H), jnp.float32) for _ in range(NBUF)],
            [pltpu.SemaphoreType.DMA for _ in range(NBUF)],
            [pltpu.SemaphoreType.DMA for _ in range(NBUF)],
            pltpu.VMEM_SHARED((NPAD, H), jnp.float32),
        ],
    )


SLAB = NPAD // NW


def _pool_body(p_hbm, hs_hbm, disb_hbm, batch_hbm, ninf_hbm, out_hbm,
               p0_v, p1_v, hs_v, disb_v, batch_v, pooled_v, lsem):
    c = lax.axis_index("c")
    s = lax.axis_index("s")
    wid = c * NS + s
    base = wid * SLAB
    cps = [
        pltpu.make_async_copy(p_hbm.at[0, pl.ds(base, SLAB)], p0_v, lsem),
        pltpu.make_async_copy(p_hbm.at[1, pl.ds(base, SLAB)], p1_v, lsem),
        pltpu.make_async_copy(hs_hbm.at[pl.ds(base, SLAB)], hs_v, lsem),
        pltpu.make_async_copy(disb_hbm.at[pl.ds(base, SLAB)], disb_v, lsem),
        pltpu.make_async_copy(batch_hbm.at[pl.ds(base, SLAB)], batch_v, lsem),
        pltpu.make_async_copy(ninf_hbm, pooled_v, lsem),
    ]
    for cp in cps:
        cp.start()
    for cp in cps:
        cp.wait()

    def vec_body(d, carry):
        for k in range(2):
            f = pl.ds(16 * k, 16)
            hs_v[d, f] = (
                p0_v[d, f] + p1_v[d, f] + hs_v[d, f]) * disb_v[d, f]
        return carry
    lax.fori_loop(0, SLAB, vec_body, 0)

    def row_body(i, carry):
        gvec = batch_v[pl.ds(i * 16, 16)]
        for k in range(16):
            g = gvec[k]

            @pl.when(g >= 0)
            def _(g=g, k=k):
                d = i * 16 + k
                for f in range(2):
                    sl = pl.ds(16 * f, 16)
                    pooled_v[g, sl] = jnp.maximum(pooled_v[g, sl],
                                                  hs_v[d, sl])
        return carry
    lax.fori_loop(0, SLAB // 16, row_body, 0)

    pltpu.sync_copy(pooled_v, out_hbm.at[wid])


@functools.lru_cache(maxsize=None)
def _pool_kernel():
    return pl.kernel(
        _pool_body,
        out_type=jax.ShapeDtypeStruct((NW, G, H), jnp.float32),
        mesh=_mesh(),
        compiler_params=_SC_PARAMS,
        scratch_types=[
            pltpu.VMEM((SLAB, H), jnp.float32),
            pltpu.VMEM((SLAB, H), jnp.float32),
            pltpu.VMEM((SLAB, H), jnp.float32),
            pltpu.VMEM((SLAB, H), jnp.float32),
            pltpu.VMEM((SLAB,), jnp.int32),
            pltpu.VMEM((G, H), jnp.float32),
            pltpu.SemaphoreType.DMA,
        ],
    )


def _head_body(pooledp_ref, b1_ref,
               wc1_ref, bc1_ref, wc2_ref, bc2_ref, wc3_ref, bc3_ref,
               out_ref):
    m = jnp.max(pooledp_ref[...], axis=0)
    pooled = jnp.where(jnp.isneginf(m), 0.0, m + b1_ref[...])
    z = jnp.maximum(
        jnp.dot(pooled, wc1_ref[...], preferred_element_type=jnp.float32)
        + bc1_ref[...], 0.0)
    z = jnp.maximum(
        jnp.dot(z, wc2_ref[...], preferred_element_type=jnp.float32)
        + bc2_ref[...], 0.0)
    out_ref[...] = (
        jnp.dot(z, wc3_ref[...], preferred_element_type=jnp.float32)
        + bc3_ref[...])


def _head(pooledp, b1, wc1, bc1, wc2, bc2, wc3, bc3):
    return pl.pallas_call(
        _head_body,
        out_shape=jax.ShapeDtypeStruct((G, 4), jnp.float32),
    )(pooledp, b1, wc1, bc1, wc2, bc2, wc3, bc3)


def kernel(x, edge_index, batch, W1, b1, Wc1, bc1, Wc2, bc2, Wc3, bc3):
    e4 = edge_index.reshape(2, NW, NCHUNK, CH)

    ones_rows = jnp.ones((CH, DW), jnp.float32)
    zeros_deg = jnp.zeros((NPT, DW), jnp.float32)
    zeros_acc = jnp.zeros((NPT, H), jnp.float32)
    ninf = jnp.full((G, H), -jnp.inf, jnp.float32)
    batch_pad = jnp.pad(batch, (0, NPAD - N), constant_values=-1)

    h = _mma(x, W1)
    degb = _deg_kernel()(e4, ones_rows, zeros_deg)
    hs4, disb4 = _mmb(h.reshape(NF, 128), degb)
    hs = hs4.reshape(NPAD, H)
    disb = disb4.reshape(NPAD, H)
    parts = _agg_kernel()(e4, hs, zeros_acc)
    pooledp = _pool_kernel()(parts, hs, disb, batch_pad, ninf)
    out = _head(pooledp, b1.reshape(1, H),
                Wc1, bc1.reshape(1, -1), Wc2, bc2.reshape(1, -1),
                Wc3, bc3.reshape(1, -1))
    return out


# speedup vs baseline: 75.6022x; 1.0170x over previous
import functools

import jax
import jax.numpy as jnp
from jax import lax
from jax.experimental import pallas as pl
from jax.experimental.pallas import tpu as pltpu
from jax.experimental.pallas import tpu_sc as plsc

N = 10000
E = 320000
D = 128
H = 32
G = 64

NC = 2
NS = 16
NW = NC * NS
EPT = E // NW
CH = 125
NCHUNK = EPT // CH
NPAD = 10240
NPT = NPAD // NS
DW = 16


def _mesh():
    return plsc.VectorSubcoreMesh(
        core_axis_name="c", subcore_axis_name="s",
        num_cores=NC, num_subcores=NS)


_SC_PARAMS = pltpu.CompilerParams(use_tc_tiling_on_sc=False)


def _deg_body(e_hbm, ones_hbm, zeros_hbm, out_hbm, dst_v, ones_v, dsem,
              deg_v, degb_v, deg_sh):
    c = lax.axis_index("c")
    s = lax.axis_index("s")
    wid = c * NS + s
    pltpu.sync_copy(e_hbm.at[1, wid], dst_v)
    pltpu.sync_copy(ones_hbm, ones_v)
    pltpu.sync_copy(zeros_hbm, deg_sh.at[pl.ds(s * NPT, NPT)])
    plsc.subcore_barrier()

    def chunk_body(j, carry):
        pltpu.async_copy(ones_v, deg_sh.at[dst_v.at[j]], dsem, add=True)
        return carry
    lax.fori_loop(0, NCHUNK, chunk_body, 0)

    def drain_body(j, carry):
        pltpu.make_async_copy(ones_v, deg_sh.at[dst_v.at[j]], dsem).wait()
        return carry
    lax.fori_loop(0, NCHUNK, drain_body, 0)

    plsc.subcore_barrier()
    pltpu.sync_copy(deg_sh.at[pl.ds(s * NPT, NPT)], deg_v)

    def splat_body(n, carry):
        val = deg_v[n]
        flat = n * H
        row = flat // 128
        col = flat % 128
        degb_v[row, pl.ds(col, 16)] = val
        degb_v[row, pl.ds(col + 16, 16)] = val
        return carry
    lax.fori_loop(0, NPT, splat_body, 0)
    pltpu.sync_copy(degb_v, out_hbm.at[c, pl.ds(s * (NPT * H // 128),
                                                NPT * H // 128)])


@functools.lru_cache(maxsize=None)
def _deg_kernel():
    return pl.kernel(
        _deg_body,
        out_type=jax.ShapeDtypeStruct((NC, NPAD * H // 128, 128),
                                      jnp.float32),
        mesh=_mesh(),
        compiler_params=_SC_PARAMS,
        scratch_types=[
            pltpu.VMEM((NCHUNK, CH), jnp.int32),
            pltpu.VMEM((CH, DW), jnp.float32),
            pltpu.SemaphoreType.DMA,
            pltpu.VMEM((NPT, DW), jnp.float32),
            pltpu.VMEM((NPT * H // 128, 128), jnp.float32),
            pltpu.VMEM_SHARED((NPAD, DW), jnp.float32),
        ],
    )


def _mma_body(x_ref, w_ref, h_ref):
    h_ref[pl.ds(0, N)] = jnp.dot(x_ref[...], w_ref[...],
                                 preferred_element_type=jnp.float32)
    h_ref[pl.ds(N, NPAD - N)] = jnp.zeros((NPAD - N, H), jnp.float32)


def _mma(x, w1):
    return pl.pallas_call(
        _mma_body,
        out_shape=jax.ShapeDtypeStruct((NPAD, H), jnp.float32),
    )(x, w1)


NF = NPAD * H // 128


def _mmb_body(h_ref, degb_ref, hs_ref, disb_ref):
    deg = degb_ref[0] + degb_ref[1] + 1.0
    disc = lax.rsqrt(deg)
    hs_ref[...] = h_ref[...] * disc
    disb_ref[...] = disc


def _mmb(h4, degb):
    return pl.pallas_call(
        _mmb_body,
        out_shape=[
            jax.ShapeDtypeStruct((NF, 128), jnp.float32),
            jax.ShapeDtypeStruct((NF, 128), jnp.float32),
        ],
    )(h4, degb)


NBUF = 10
PD = 5


def _agg_body(e_hbm, hs_hbm, zeros_hbm, out_hbm,
              src_v, dst_v, rows_v, gsems, ssems, acc_sh):
    c = lax.axis_index("c")
    s = lax.axis_index("s")
    wid = c * NS + s
    pltpu.sync_copy(e_hbm.at[0, wid], src_v)
    pltpu.sync_copy(e_hbm.at[1, wid], dst_v)
    pltpu.sync_copy(zeros_hbm, acc_sh.at[pl.ds(s * NPT, NPT)])
    plsc.subcore_barrier()

    for b in range(PD):
        pltpu.async_copy(hs_hbm.at[src_v.at[b]], rows_v[b], gsems[b])

    def ring_body(i, carry):
        for k in range(NBUF):
            j = NBUF * i + k
            bp = (k + PD) % NBUF
            pltpu.make_async_copy(hs_hbm.at[src_v.at[j]], rows_v[k],
                                  gsems[k]).wait()
            pltpu.async_copy(rows_v[k], acc_sh.at[dst_v.at[j]], ssems[k],
                             add=True)

            @pl.when((j + PD < NCHUNK) & (j >= NBUF - PD))
            def _(j=j, bp=bp):
                pltpu.make_async_copy(rows_v[bp],
                                      acc_sh.at[dst_v.at[j - (NBUF - PD)]],
                                      ssems[bp]).wait()

            @pl.when(j + PD < NCHUNK)
            def _(j=j, bp=bp):
                pltpu.async_copy(hs_hbm.at[src_v.at[j + PD]], rows_v[bp],
                                 gsems[bp])
        return carry
    lax.fori_loop(0, NCHUNK // NBUF, ring_body, 0)

    for k in range(NBUF):
        pltpu.make_async_copy(rows_v[k],
                              acc_sh.at[dst_v.at[NCHUNK - NBUF + k]],
                              ssems[k]).wait()

    plsc.subcore_barrier()
    pltpu.sync_copy(acc_sh.at[pl.ds(s * NPT, NPT)],
                    out_hbm.at[c, pl.ds(s * NPT, NPT)])


@functools.lru_cache(maxsize=None)
def _agg_kernel():
    return pl.kernel(
        _agg_body,
        out_type=jax.ShapeDtypeStruct((NC, NPAD, H), jnp.float32),
        mesh=_mesh(),
        compiler_params=_SC_PARAMS,
        scratch_types=[
            pltpu.VMEM((NCHUNK, CH), jnp.int32),
            pltpu.VMEM((NCHUNK, CH), jnp.int32),
            [pltpu.VMEM((CH, H), jnp.float32) for _ in range(NBUF)],
            [pltpu.SemaphoreType.DMA for _ in range(NBUF)],
            [pltpu.SemaphoreType.DMA for _ in range(NBUF)],
            pltpu.VMEM_SHARED((NPAD, H), jnp.float32),
        ],
    )


SLAB = NPAD // NW


def _pool_body(p_hbm, hs_hbm, disb_hbm, batch_hbm, ninf_hbm, out_hbm,
               p0_v, p1_v, hs_v, disb_v, batch_v, pooled_v, lsem):
    c = lax.axis_index("c")
    s = lax.axis_index("s")
    wid = c * NS + s
    base = wid * SLAB
    cps = [
        pltpu.make_async_copy(p_hbm.at[0, pl.ds(base, SLAB)], p0_v, lsem),
        pltpu.make_async_copy(p_hbm.at[1, pl.ds(base, SLAB)], p1_v, lsem),
        pltpu.make_async_copy(hs_hbm.at[pl.ds(base, SLAB)], hs_v, lsem),
        pltpu.make_async_copy(disb_hbm.at[pl.ds(base, SLAB)], disb_v, lsem),
        pltpu.make_async_copy(batch_hbm.at[pl.ds(base, SLAB)], batch_v, lsem),
        pltpu.make_async_copy(ninf_hbm, pooled_v, lsem),
    ]
    for cp in cps:
        cp.start()
    for cp in cps:
        cp.wait()

    def vec_body(d, carry):
        for k in range(2):
            f = pl.ds(16 * k, 16)
            hs_v[d, f] = (
                p0_v[d, f] + p1_v[d, f] + hs_v[d, f]) * disb_v[d, f]
        return carry
    lax.fori_loop(0, SLAB, vec_body, 0)

    def row_body(i, carry):
        gvec = batch_v[pl.ds(i * 16, 16)]
        for k in range(16):
            g = gvec[k]

            @pl.when(g >= 0)
            def _(g=g, k=k):
                d = i * 16 + k
                for f in range(2):
                    sl = pl.ds(16 * f, 16)
                    pooled_v[g, sl] = jnp.maximum(pooled_v[g, sl],
                                                  hs_v[d, sl])
        return carry
    lax.fori_loop(0, SLAB // 16, row_body, 0)

    pltpu.sync_copy(pooled_v, out_hbm.at[wid])


@functools.lru_cache(maxsize=None)
def _pool_kernel():
    return pl.kernel(
        _pool_body,
        out_type=jax.ShapeDtypeStruct((NW, G, H), jnp.float32),
        mesh=_mesh(),
        compiler_params=_SC_PARAMS,
        scratch_types=[
            pltpu.VMEM((SLAB, H), jnp.float32),
            pltpu.VMEM((SLAB, H), jnp.float32),
            pltpu.VMEM((SLAB, H), jnp.float32),
            pltpu.VMEM((SLAB, H), jnp.float32),
            pltpu.VMEM((SLAB,), jnp.int32),
            pltpu.VMEM((G, H), jnp.float32),
            pltpu.SemaphoreType.DMA,
        ],
    )


def _head_body(pooledp_ref, b1_ref,
               wc1_ref, bc1_ref, wc2_ref, bc2_ref, wc3_ref, bc3_ref,
               out_ref):
    m = jnp.max(pooledp_ref[...], axis=0)
    pooled = jnp.where(jnp.isneginf(m), 0.0, m + b1_ref[...])
    z = jnp.maximum(
        jnp.dot(pooled, wc1_ref[...], preferred_element_type=jnp.float32)
        + bc1_ref[...], 0.0)
    z = jnp.maximum(
        jnp.dot(z, wc2_ref[...], preferred_element_type=jnp.float32)
        + bc2_ref[...], 0.0)
    out_ref[...] = (
        jnp.dot(z, wc3_ref[...], preferred_element_type=jnp.float32)
        + bc3_ref[...])


def _head(pooledp, b1, wc1, bc1, wc2, bc2, wc3, bc3):
    return pl.pallas_call(
        _head_body,
        out_shape=jax.ShapeDtypeStruct((G, 4), jnp.float32),
    )(pooledp, b1, wc1, bc1, wc2, bc2, wc3, bc3)


def kernel(x, edge_index, batch, W1, b1, Wc1, bc1, Wc2, bc2, Wc3, bc3):
    e4 = edge_index.reshape(2, NW, NCHUNK, CH)

    ones_rows = jnp.ones((CH, DW), jnp.float32)
    zeros_deg = jnp.zeros((NPT, DW), jnp.float32)
    zeros_acc = jnp.zeros((NPT, H), jnp.float32)
    ninf = jnp.full((G, H), -jnp.inf, jnp.float32)
    batch_pad = jnp.pad(batch, (0, NPAD - N), constant_values=-1)

    h = _mma(x, W1)
    degb = _deg_kernel()(e4, ones_rows, zeros_deg)
    hs4, disb4 = _mmb(h.reshape(NF, 128), degb)
    hs = hs4.reshape(NPAD, H)
    disb = disb4.reshape(NPAD, H)
    parts = _agg_kernel()(e4, hs, zeros_acc)
    pooledp = _pool_kernel()(parts, hs, disb, batch_pad, ninf)
    out = _head(pooledp, b1.reshape(1, H),
                Wc1, bc1.reshape(1, -1), Wc2, bc2.reshape(1, -1),
                Wc3, bc3.reshape(1, -1))
    return out
